# baseline (speedup 1.0000x reference)
import jax
import jax.numpy as jnp
from jax.experimental import pallas as pl
from jax.experimental.pallas import tpu as pltpu

_B, _CIN, _COUT, _H, _W = 4, 96, 96, 224, 224
_R = 56
_WP = _W + 2
_NR = _H // _R
_TOT = _B * _NR
_NS = 3


def _row_copy(x_hbm, xs_ref, sem, bb, h, sl, j):
    return pltpu.make_async_copy(
        x_hbm.at[bb, :, h, :],
        xs_ref.at[sl, j],
        sem.at[sl])


def _issue(x_hbm, xs_ref, sem, bb, rr, sl):
    base = rr * _R - 1
    for j in range(_R + 2):
        if j == 0:
            @pl.when(rr == 0)
            def _():
                xs_ref[sl, 0] = jnp.zeros((_CIN, _W), jnp.float32)

            @pl.when(rr > 0)
            def _():
                _row_copy(x_hbm, xs_ref, sem, bb, base, sl, 0).start()
        elif j == _R + 1:
            @pl.when(rr == _NR - 1)
            def _():
                xs_ref[sl, _R + 1] = jnp.zeros((_CIN, _W), jnp.float32)

            @pl.when(rr < _NR - 1)
            def _():
                _row_copy(x_hbm, xs_ref, sem, bb, base + _R + 1, sl,
                          _R + 1).start()
        else:
            _row_copy(x_hbm, xs_ref, sem, bb, base + j, sl, j).start()


def _wait(x_hbm, xs_ref, sem, bb, rr, sl):
    base = rr * _R - 1
    for j in range(_R + 2):
        if j == 0:
            @pl.when(rr > 0)
            def _():
                _row_copy(x_hbm, xs_ref, sem, bb, base, sl, 0).wait()
        elif j == _R + 1:
            @pl.when(rr < _NR - 1)
            def _():
                _row_copy(x_hbm, xs_ref, sem, bb, base + _R + 1, sl,
                          _R + 1).wait()
        else:
            _row_copy(x_hbm, xs_ref, sem, bb, base + j, sl, j).wait()


def _out_copy(o_hbm, os_ref, osem, bb, rr, sl, i):
    return pltpu.make_async_copy(
        os_ref.at[sl, i],
        o_hbm.at[bb, :, rr * _R + i, :],
        osem.at[sl])


def _conv_body(w_ref, b_ref, x_hbm, o_hbm, xs_ref, os_ref, sem, osem):
    b = pl.program_id(0)
    r = pl.program_id(1)
    step = b * _NR + r
    slot = jax.lax.rem(step, _NS)
    oslot = jax.lax.rem(step, 2)

    @pl.when(step == 0)
    def _():
        _issue(x_hbm, xs_ref, sem, 0, 0, 0)
        _issue(x_hbm, xs_ref, sem, 0, 1, 1)

    nb = jnp.where(r + 2 >= _NR, b + 1, b)
    nr = jnp.where(r + 2 >= _NR, r + 2 - _NR, r + 2)

    @pl.when(step + 2 < _TOT)
    def _():
        _issue(x_hbm, xs_ref, sem, nb, nr, jax.lax.rem(step + 2, _NS))

    pb = jnp.where(r >= 2, b, b - 1)
    pr = jnp.where(r >= 2, r - 2, r - 2 + _NR)

    @pl.when(step >= 2)
    def _():
        for i in range(_R):
            _out_copy(o_hbm, os_ref, osem, pb, pr, oslot, i).wait()

    _wait(x_hbm, xs_ref, sem, b, r, slot)

    xwin = xs_ref[slot].astype(jnp.bfloat16)
    xcat = xwin.reshape((_R + 2) * _CIN, _W)
    sh = [jnp.pad(xcat[:, :_W - 1], ((0, 0), (1, 0))),
          xcat,
          jnp.pad(xcat[:, 1:], ((0, 0), (0, 1)))]
    w = w_ref[...]
    bias = b_ref[...]
    for i in range(_R):
        acc = jnp.zeros((_COUT, _W), jnp.float32)
        for kw in range(3):
            acc += jnp.dot(w[kw], sh[kw][i * _CIN:(i + 3) * _CIN],
                           preferred_element_type=jnp.float32)
        os_ref[oslot, i] = jnp.clip(acc + bias, -10000.0, 10000.0)

    for i in range(_R):
        _out_copy(o_hbm, os_ref, osem, b, r, oslot, i).start()

    @pl.when(step == _TOT - 1)
    def _():
        for i in range(_R):
            _out_copy(o_hbm, os_ref, osem, b, jnp.where(r > 0, r - 1, 0),
                      1 - oslot, i).wait()
        for i in range(_R):
            _out_copy(o_hbm, os_ref, osem, b, r, oslot, i).wait()


def kernel(x, weight, bias, scale):
    w_eff = jnp.sign(weight) * scale
    w3 = jnp.transpose(w_eff, (3, 0, 2, 1)).reshape(3, _COUT, 3 * _CIN)
    w3 = w3.astype(jnp.bfloat16)
    grid = (_B, _NR)
    out = pl.pallas_call(
        _conv_body,
        grid=grid,
        in_specs=[
            pl.BlockSpec((3, _COUT, 3 * _CIN), lambda b, r: (0, 0, 0)),
            pl.BlockSpec((_COUT, 1), lambda b, r: (0, 0)),
            pl.BlockSpec(memory_space=pl.ANY),
        ],
        out_specs=pl.BlockSpec(memory_space=pl.ANY),
        out_shape=jax.ShapeDtypeStruct((_B, _COUT, _H, _W), jnp.float32),
        scratch_shapes=[
            pltpu.VMEM((_NS, _R + 2, _CIN, _W), jnp.float32),
            pltpu.VMEM((2, _R, _COUT, _W), jnp.float32),
            pltpu.SemaphoreType.DMA((_NS,)),
            pltpu.SemaphoreType.DMA((2,)),
        ],
    )(w3, bias.reshape(_COUT, 1), x)
    return out


# speedup vs baseline: 1.0319x; 1.0319x over previous
import jax
import jax.numpy as jnp
from jax.experimental import pallas as pl
from jax.experimental.pallas import tpu as pltpu

_B, _CIN, _COUT, _H, _W = 4, 96, 96, 224, 224
_R = 56
_WP = _W + 2
_NR = _H // _R
_TOT = _B * _NR
_NS = 3


def _row_copy(x_hbm, xs_ref, sem, bb, h, sl, j):
    return pltpu.make_async_copy(
        x_hbm.at[bb, :, h, :],
        xs_ref.at[sl, j],
        sem.at[sl])


def _issue(x_hbm, xs_ref, sem, bb, rr, sl):
    base = rr * _R - 1
    for j in range(_R + 2):
        if j == 0:
            @pl.when(rr == 0)
            def _():
                xs_ref[sl, 0] = jnp.zeros((_CIN, _W), jnp.float32)

            @pl.when(rr > 0)
            def _():
                _row_copy(x_hbm, xs_ref, sem, bb, base, sl, 0).start()
        elif j == _R + 1:
            @pl.when(rr == _NR - 1)
            def _():
                xs_ref[sl, _R + 1] = jnp.zeros((_CIN, _W), jnp.float32)

            @pl.when(rr < _NR - 1)
            def _():
                _row_copy(x_hbm, xs_ref, sem, bb, base + _R + 1, sl,
                          _R + 1).start()
        else:
            _row_copy(x_hbm, xs_ref, sem, bb, base + j, sl, j).start()


def _wait(x_hbm, xs_ref, sem, bb, rr, sl):
    base = rr * _R - 1
    for j in range(_R + 2):
        if j == 0:
            @pl.when(rr > 0)
            def _():
                _row_copy(x_hbm, xs_ref, sem, bb, base, sl, 0).wait()
        elif j == _R + 1:
            @pl.when(rr < _NR - 1)
            def _():
                _row_copy(x_hbm, xs_ref, sem, bb, base + _R + 1, sl,
                          _R + 1).wait()
        else:
            _row_copy(x_hbm, xs_ref, sem, bb, base + j, sl, j).wait()


def _out_copy(o_hbm, os_ref, osem, bb, rr, sl, i):
    return pltpu.make_async_copy(
        os_ref.at[sl, i],
        o_hbm.at[bb, :, rr * _R + i, :],
        osem.at[sl])


def _conv_body(w_ref, b_ref, x_hbm, o_hbm, xs_ref, os_ref, sem, osem):
    b = pl.program_id(0)
    r = pl.program_id(1)
    step = b * _NR + r
    slot = jax.lax.rem(step, _NS)
    oslot = jax.lax.rem(step, 2)

    @pl.when(step == 0)
    def _():
        _issue(x_hbm, xs_ref, sem, 0, 0, 0)
        _issue(x_hbm, xs_ref, sem, 0, 1, 1)

    nb = jnp.where(r + 2 >= _NR, b + 1, b)
    nr = jnp.where(r + 2 >= _NR, r + 2 - _NR, r + 2)

    @pl.when(step + 2 < _TOT)
    def _():
        _issue(x_hbm, xs_ref, sem, nb, nr, jax.lax.rem(step + 2, _NS))

    pb = jnp.where(r >= 2, b, b - 1)
    pr = jnp.where(r >= 2, r - 2, r - 2 + _NR)

    @pl.when(step >= 2)
    def _():
        for i in range(_R):
            _out_copy(o_hbm, os_ref, osem, pb, pr, oslot, i).wait()

    _wait(x_hbm, xs_ref, sem, b, r, slot)

    xwin = xs_ref[slot].astype(jnp.bfloat16)
    xcat = jnp.pad(xwin.reshape((_R + 2) * _CIN, _W), ((0, 0), (1, 1)))
    sh = [xcat[:, kw:kw + _W] for kw in range(3)]
    w = w_ref[...]
    bias = b_ref[...]
    for i in range(_R):
        acc = jnp.zeros((_COUT, _W), jnp.float32)
        for kw in range(3):
            acc += jnp.dot(w[kw], sh[kw][i * _CIN:(i + 3) * _CIN],
                           preferred_element_type=jnp.float32)
        os_ref[oslot, i] = jnp.clip(acc + bias, -10000.0, 10000.0)

    for i in range(_R):
        _out_copy(o_hbm, os_ref, osem, b, r, oslot, i).start()

    @pl.when(step == _TOT - 1)
    def _():
        for i in range(_R):
            _out_copy(o_hbm, os_ref, osem, b, jnp.where(r > 0, r - 1, 0),
                      1 - oslot, i).wait()
        for i in range(_R):
            _out_copy(o_hbm, os_ref, osem, b, r, oslot, i).wait()


def kernel(x, weight, bias, scale):
    w_eff = jnp.sign(weight) * scale
    w3 = jnp.transpose(w_eff, (3, 0, 2, 1)).reshape(3, _COUT, 3 * _CIN)
    w3 = w3.astype(jnp.bfloat16)
    grid = (_B, _NR)
    out = pl.pallas_call(
        _conv_body,
        grid=grid,
        in_specs=[
            pl.BlockSpec((3, _COUT, 3 * _CIN), lambda b, r: (0, 0, 0)),
            pl.BlockSpec((_COUT, 1), lambda b, r: (0, 0)),
            pl.BlockSpec(memory_space=pl.ANY),
        ],
        out_specs=pl.BlockSpec(memory_space=pl.ANY),
        out_shape=jax.ShapeDtypeStruct((_B, _COUT, _H, _W), jnp.float32),
        scratch_shapes=[
            pltpu.VMEM((_NS, _R + 2, _CIN, _W), jnp.float32),
            pltpu.VMEM((2, _R, _COUT, _W), jnp.float32),
            pltpu.SemaphoreType.DMA((_NS,)),
            pltpu.SemaphoreType.DMA((2,)),
        ],
    )(w3, bias.reshape(_COUT, 1), x)
    return out
